# Initial kernel scaffold; baseline (speedup 1.0000x reference)
import jax
import jax.numpy as jnp
from jax.experimental import pallas as pl


def kernel(era5, basinContinuous, basinDiscrete, riverContinuous, riverDiscrete, bp_Wc, bp_bc, bp_Wd, bp_bd, g1_W, g1_as, g1_ad, g1_b, g2_W, g2_as, g2_ad, g2_b, rp_Wc, rp_bc, rp_Wd, rp_bd, W_ih, W_hh, b_lstm, head_W, head_b, edge_index, nodes):
    raise NotImplementedError("write your pallas kernel here")


# speedup vs baseline: 4.0209x; 4.0209x over previous
import functools
import jax
import jax.numpy as jnp
from jax.experimental import pallas as pl
from jax.experimental.pallas import tpu as pltpu

N = 10000; T = 16; E = 160000; B = 8
D_ERA = 16; D_BC = 32; D_BD = 16; D_RC = 16; D_RD = 8
H = 128; LH = 256; K = 3


def _lstm_head_body(series_ref, wih_ref, whh_ref, b_ref, hw_ref, hb_ref,
                    cast_ref, h_ref, c_ref):
    h = jnp.zeros((B, LH), jnp.float32)
    c = jnp.zeros((B, LH), jnp.float32)
    wih = wih_ref[...]
    whh = whh_ref[...]
    b = b_ref[...]
    hw = hw_ref[...]
    hb = hb_ref[...]
    for t in range(T):
        x_t = series_ref[t]
        z = jnp.dot(x_t, wih, preferred_element_type=jnp.float32) + \
            jnp.dot(h, whh, preferred_element_type=jnp.float32) + b
        i = jax.nn.sigmoid(z[:, 0 * LH:1 * LH])
        f = jax.nn.sigmoid(z[:, 1 * LH:2 * LH])
        g = jnp.tanh(z[:, 2 * LH:3 * LH])
        o = jax.nn.sigmoid(z[:, 3 * LH:4 * LH])
        c = f * c + i * g
        h = o * jnp.tanh(c)
        zc = jnp.dot(h, hw, preferred_element_type=jnp.float32) + hb
        m_ = zc[:, 0:K]
        b_ = jax.nn.softplus(zc[:, K:2 * K]) + 1e-5
        t_ = jax.nn.sigmoid(zc[:, 2 * K:3 * K])
        p_ = jax.nn.softmax(zc[:, 3 * K:4 * K], axis=-1)
        cast_ref[t] = jnp.concatenate([m_, b_, t_, p_], axis=-1)
    h_ref[...] = h
    c_ref[...] = c


def _lstm_head(series_tbh, W_ih, W_hh, b_lstm, head_W, head_b):
    return pl.pallas_call(
        _lstm_head_body,
        out_shape=(
            jax.ShapeDtypeStruct((T, B, 4 * K), jnp.float32),
            jax.ShapeDtypeStruct((B, LH), jnp.float32),
            jax.ShapeDtypeStruct((B, LH), jnp.float32),
        ),
    )(series_tbh, W_ih, W_hh, b_lstm, head_W, head_b)


def kernel(era5, basinContinuous, basinDiscrete, riverContinuous, riverDiscrete,
           bp_Wc, bp_bc, bp_Wd, bp_bd,
           g1_W, g1_as, g1_ad, g1_b, g2_W, g2_as, g2_ad, g2_b,
           rp_Wc, rp_bc, rp_Wd, rp_bd,
           W_ih, W_hh, b_lstm, head_W, head_b,
           edge_index, nodes):
    src, dst = edge_index[0], edge_index[1]

    W_era = bp_Wc[:D_ERA]
    W_bc = bp_Wc[D_ERA:]
    base = basinContinuous @ W_bc + basinDiscrete @ bp_Wd + (bp_bc + bp_bd)
    x_all = jax.nn.relu(jnp.einsum('ntd,dh->tnh', era5, W_era) + base[None])

    v1s = g1_W @ g1_as
    v1d = g1_W @ g1_ad
    s1s = x_all @ v1s
    s1d = x_all @ v1d
    M1 = jnp.max(s1s, axis=1) + jnp.max(s1d, axis=1)

    def layer1_t(x_t, s1s_t, s1d_t, M1_t):
        e_raw = s1s_t[src] + s1d_t[dst]
        e_act = jnp.where(e_raw >= 0, e_raw, 0.2 * e_raw)
        ex = jnp.exp(e_act - M1_t)
        denom = jax.ops.segment_sum(ex, dst, num_segments=N)
        P1 = jax.ops.segment_sum(ex[:, None] * x_t[src], dst, num_segments=N)
        return P1, denom

    P1, denom = jax.vmap(layer1_t)(x_all, s1s, s1d, M1)
    h1 = jax.nn.elu((P1 / (denom[..., None] + 1e-16)) @ g1_W + g1_b)

    batchIndices = jnp.concatenate([jnp.zeros((1,), nodes.dtype), jnp.cumsum(nodes)[:-1]])
    eq = dst[:, None] == batchIndices[None, :]
    has = jnp.any(eq, axis=1)
    slot = jnp.where(has, jnp.argmax(eq, axis=1), 8)

    v2s = g2_W @ g2_as
    v2d = g2_W @ g2_ad
    s2s = h1 @ v2s
    s2d_sel = h1[:, batchIndices, :] @ v2d
    M2 = jnp.max(s2s, axis=1) + jnp.max(s2d_sel, axis=1)

    def layer2_t(h1_t, s2s_t, s2d_sel_t, M2_t):
        s2d_e = jnp.concatenate([s2d_sel_t, jnp.zeros((1,), jnp.float32)])[slot]
        e_raw = s2s_t[src] + s2d_e
        e_act = jnp.where(e_raw >= 0, e_raw, 0.2 * e_raw)
        ex = jnp.where(has, jnp.exp(e_act - M2_t), 0.0)
        denom2 = jax.ops.segment_sum(ex, slot, num_segments=9)[:8]
        agg = jax.ops.segment_sum(ex[:, None] * h1_t[src], slot, num_segments=9)[:8]
        return (agg / (denom2[:, None] + 1e-16)) @ g2_W + g2_b

    out2 = jax.vmap(layer2_t)(h1, s2s, s2d_sel, M2)
    first = jnp.argmax(batchIndices[None, :] == batchIndices[:, None], axis=1)
    out2 = out2[:, first, :]

    rcat = jnp.concatenate([out2, jnp.broadcast_to(riverContinuous[None], (T, B, D_RC))], -1)
    series = jax.nn.relu(rcat @ rp_Wc + rp_bc + riverDiscrete @ rp_Wd + rp_bd)

    cast_t, h, c = _lstm_head(series, W_ih, W_hh, b_lstm, head_W, head_b)
    cast = jnp.swapaxes(cast_t, 0, 1)
    return cast, (h, c)


# speedup vs baseline: 7.3487x; 1.8276x over previous
import functools
import jax
import jax.numpy as jnp
from jax import lax
from jax.experimental import pallas as pl
from jax.experimental.pallas import tpu as pltpu
from jax.experimental.pallas import tpu_sc as plsc

N = 10000; T = 16; E = 160000; B = 8
D_ERA = 16; D_BC = 32; D_BD = 16; D_RC = 16; D_RD = 8
H = 128; LH = 256; K = 3

NC = 2
NS = 16
EC = E // NS
ECP = 10016
NG = ECP // 16
ECA = ECP + 16
HC = 80
NP = 10240
NPT = NP // NS


def _sc1_body(xA, xB, s1s, s1d, m1, srcp, dstp,
              p1parts,
              srcc, dstc, s1sv, s1dv, m1v,
              idxS0, idxS1, idxD, rows0, rows1, scaled,
              zb, sem0, sem1,
              p1acc):
    c = lax.axis_index("c")
    s = lax.axis_index("s")
    z16 = jnp.zeros((16,), jnp.float32)
    iota = lax.iota(jnp.int32, 16)

    pltpu.sync_copy(srcp.at[s], srcc)
    pltpu.sync_copy(dstp.at[s], dstc)

    def zero_zb(i, _):
        for k in range(HC // 16):
            zb[i, pl.ds(k * 16, 16)] = z16
        return _
    lax.fori_loop(0, 320, zero_zb, None)

    def issue(g, toff, idxSb, rowsb, semb):
        src16 = srcc[pl.ds(g * 16, 16)]
        idxSb[...] = src16 + toff
        @pl.when(c == 0)
        def _():
            pltpu.async_copy(xA.at[idxSb], rowsb, semb)
        @pl.when(c == 1)
        def _():
            pltpu.async_copy(xB.at[idxSb], rowsb, semb)

    def wait(idxSb, rowsb, semb):
        @pl.when(c == 0)
        def _():
            pltpu.make_async_copy(xA.at[idxSb], rowsb, semb).wait()
        @pl.when(c == 1)
        def _():
            pltpu.make_async_copy(xB.at[idxSb], rowsb, semb).wait()

    def t_body(t, _):
        pltpu.sync_copy(s1s.at[t], s1sv)
        pltpu.sync_copy(s1d.at[t], s1dv)
        pltpu.sync_copy(m1.at[t], m1v)
        m1t = m1v[...]
        toff = t * N

        for j in range(2):
            dst_off = pl.multiple_of(s * NPT + j * 320, 8)
            pltpu.sync_copy(zb, p1acc.at[pl.ds(dst_off, 320), :])
        plsc.subcore_barrier()

        issue(0, toff, idxS0, rows0, sem0)

        def process(g, idxSb, rowsb, semb):
            src16 = srcc[pl.ds(g * 16, 16)]
            dst16 = dstc[pl.ds(g * 16, 16)]
            s_s = plsc.load_gather(s1sv, [src16])
            s_d = plsc.load_gather(s1dv, [dst16])
            e = s_s + s_d
            e = jnp.where(e >= 0.0, e, e * 0.2)
            ex = jnp.exp(e - m1t)
            ex = jnp.where((g * 16 + iota) < EC, ex, 0.0)
            wait(idxSb, rowsb, semb)
            for j in range(16):
                exj = ex[j]
                for k in range(HC // 16):
                    sl = pl.ds(k * 16, 16)
                    scaled[j, sl] = rowsb[j, sl] * exj
            idxD[...] = dst16
            pltpu.sync_copy(scaled, p1acc.at[idxD], add=True)

        def pair_body(p, _):
            g0 = p * 2
            issue(g0 + 1, toff, idxS1, rows1, sem1)
            process(g0, idxS0, rows0, sem0)
            issue(g0 + 2, toff, idxS0, rows0, sem0)
            process(g0 + 1, idxS1, rows1, sem1)
            return _
        lax.fori_loop(0, NG // 2, pair_body, None)
        wait(idxS0, rows0, sem0)
        plsc.subcore_barrier()

        src_off = pl.multiple_of(s * NPT, 8)
        pltpu.sync_copy(p1acc.at[pl.ds(src_off, NPT), :],
                        p1parts.at[c, t, pl.ds(src_off, NPT), :])
        return _
    lax.fori_loop(0, T, t_body, None)


def _sc1_layer1(xA, xB, s1s, s1d, m1, srcp, dstp):
    mesh = plsc.VectorSubcoreMesh(core_axis_name="c", subcore_axis_name="s")
    f = pl.kernel(
        _sc1_body,
        out_type=jax.ShapeDtypeStruct((NC, T, NP, HC), jnp.float32),
        mesh=mesh,
        compiler_params=pltpu.CompilerParams(needs_layout_passes=False,
                                             use_tc_tiling_on_sc=False),
        scratch_types=[
            pltpu.VMEM((ECA,), jnp.int32),
            pltpu.VMEM((ECA,), jnp.int32),
            pltpu.VMEM((N,), jnp.float32),
            pltpu.VMEM((N,), jnp.float32),
            pltpu.VMEM((16,), jnp.float32),
            pltpu.VMEM((16,), jnp.int32),
            pltpu.VMEM((16,), jnp.int32),
            pltpu.VMEM((16,), jnp.int32),
            pltpu.VMEM((16, HC), jnp.float32),
            pltpu.VMEM((16, HC), jnp.float32),
            pltpu.VMEM((16, HC), jnp.float32),
            pltpu.VMEM((320, HC), jnp.float32),
            pltpu.SemaphoreType.DMA,
            pltpu.SemaphoreType.DMA,
            pltpu.VMEM_SHARED((NP, HC), jnp.float32),
        ],
    )
    return f(xA, xB, s1s, s1d, m1, srcp, dstp)


def _lstm_head_body(series_ref, wih_ref, whh_ref, b_ref, hw_ref, hb_ref,
                    cast_ref, h_ref, c_ref):
    h = jnp.zeros((B, LH), jnp.float32)
    c = jnp.zeros((B, LH), jnp.float32)
    wih = wih_ref[...]
    whh = whh_ref[...]
    b = b_ref[...]
    hw = hw_ref[...]
    hb = hb_ref[...]
    for t in range(T):
        x_t = series_ref[t]
        z = jnp.dot(x_t, wih, preferred_element_type=jnp.float32) + \
            jnp.dot(h, whh, preferred_element_type=jnp.float32) + b
        i = jax.nn.sigmoid(z[:, 0 * LH:1 * LH])
        f = jax.nn.sigmoid(z[:, 1 * LH:2 * LH])
        g = jnp.tanh(z[:, 2 * LH:3 * LH])
        o = jax.nn.sigmoid(z[:, 3 * LH:4 * LH])
        c = f * c + i * g
        h = o * jnp.tanh(c)
        zc = jnp.dot(h, hw, preferred_element_type=jnp.float32) + hb
        m_ = zc[:, 0:K]
        b_ = jax.nn.softplus(zc[:, K:2 * K]) + 1e-5
        t_ = jax.nn.sigmoid(zc[:, 2 * K:3 * K])
        p_ = jax.nn.softmax(zc[:, 3 * K:4 * K], axis=-1)
        cast_ref[t] = jnp.concatenate([m_, b_, t_, p_], axis=-1)
    h_ref[...] = h
    c_ref[...] = c


def _lstm_head(series_tbh, W_ih, W_hh, b_lstm, head_W, head_b):
    return pl.pallas_call(
        _lstm_head_body,
        out_shape=(
            jax.ShapeDtypeStruct((T, B, 4 * K), jnp.float32),
            jax.ShapeDtypeStruct((B, LH), jnp.float32),
            jax.ShapeDtypeStruct((B, LH), jnp.float32),
        ),
    )(series_tbh, W_ih, W_hh, b_lstm, head_W, head_b)


def kernel(era5, basinContinuous, basinDiscrete, riverContinuous, riverDiscrete,
           bp_Wc, bp_bc, bp_Wd, bp_bd,
           g1_W, g1_as, g1_ad, g1_b, g2_W, g2_as, g2_ad, g2_b,
           rp_Wc, rp_bc, rp_Wd, rp_bd,
           W_ih, W_hh, b_lstm, head_W, head_b,
           edge_index, nodes):
    src, dst = edge_index[0], edge_index[1]

    W_era = bp_Wc[:D_ERA]
    W_bc = bp_Wc[D_ERA:]
    base = basinContinuous @ W_bc + basinDiscrete @ bp_Wd + (bp_bc + bp_bd)
    x_all = jax.nn.relu(jnp.einsum('ntd,dh->tnh', era5, W_era) + base[None])

    v1s = g1_W @ g1_as
    v1d = g1_W @ g1_ad
    s1s = x_all @ v1s
    s1d = x_all @ v1d
    M1 = jnp.max(s1s, axis=1) + jnp.max(s1d, axis=1)
    m1bc = jnp.broadcast_to(M1[:, None], (T, 16))

    srcp = jnp.pad(src.reshape(NS, EC), ((0, 0), (0, ECA - EC)))
    dstp = jnp.pad(dst.reshape(NS, EC), ((0, 0), (0, ECA - EC)))
    xflat = x_all.reshape(T * N, H)
    zc = jnp.zeros((T * N, 16), jnp.float32)
    xA = jnp.concatenate([xflat[:, :64], zc], axis=1)
    xB = jnp.concatenate([xflat[:, 64:], jnp.ones((T * N, 1), jnp.float32),
                          zc[:, :15]], axis=1)
    p1parts = _sc1_layer1(xA, xB, s1s, s1d, m1bc, srcp, dstp)
    P1 = jnp.concatenate([p1parts[0, :, :N, :64], p1parts[1, :, :N, :64]], -1)
    denom = p1parts[1, :, :N, 64]
    h1 = jax.nn.elu((P1 / (denom[..., None] + 1e-16)) @ g1_W + g1_b)

    batchIndices = jnp.concatenate([jnp.zeros((1,), nodes.dtype), jnp.cumsum(nodes)[:-1]])
    eq = dst[:, None] == batchIndices[None, :]
    has = jnp.any(eq, axis=1)
    slot = jnp.where(has, jnp.argmax(eq, axis=1), 8)

    v2s = g2_W @ g2_as
    v2d = g2_W @ g2_ad
    s2s = h1 @ v2s
    s2d_sel = h1[:, batchIndices, :] @ v2d
    M2 = jnp.max(s2s, axis=1) + jnp.max(s2d_sel, axis=1)

    def layer2_t(h1_t, s2s_t, s2d_sel_t, M2_t):
        s2d_e = jnp.concatenate([s2d_sel_t, jnp.zeros((1,), jnp.float32)])[slot]
        e_raw = s2s_t[src] + s2d_e
        e_act = jnp.where(e_raw >= 0, e_raw, 0.2 * e_raw)
        ex = jnp.where(has, jnp.exp(e_act - M2_t), 0.0)
        denom2 = jax.ops.segment_sum(ex, slot, num_segments=9)[:8]
        agg = jax.ops.segment_sum(ex[:, None] * h1_t[src], slot, num_segments=9)[:8]
        return (agg / (denom2[:, None] + 1e-16)) @ g2_W + g2_b

    out2 = jax.vmap(layer2_t)(h1, s2s, s2d_sel, M2)
    first = jnp.argmax(batchIndices[None, :] == batchIndices[:, None], axis=1)
    out2 = out2[:, first, :]

    rcat = jnp.concatenate([out2, jnp.broadcast_to(riverContinuous[None], (T, B, D_RC))], -1)
    series = jax.nn.relu(rcat @ rp_Wc + rp_bc + riverDiscrete @ rp_Wd + rp_bd)

    cast_t, h, c = _lstm_head(series, W_ih, W_hh, b_lstm, head_W, head_b)
    cast = jnp.swapaxes(cast_t, 0, 1)
    return cast, (h, c)


# speedup vs baseline: 26.6709x; 3.6293x over previous
import functools
import jax
import jax.numpy as jnp
from jax import lax
from jax.experimental import pallas as pl
from jax.experimental.pallas import tpu as pltpu
from jax.experimental.pallas import tpu_sc as plsc

N = 10000; T = 16; E = 160000; B = 8
D_ERA = 16; D_BC = 32; D_BD = 16; D_RC = 16; D_RD = 8
H = 128; LH = 256; K = 3

NC = 2
NS = 16
EC = E // NS
ECP = 10016
NG = ECP // 16
ECA = ECP + 16
HC = 80
NP = 10240
NPT = NP // NS


def _sc1_body(xA, xB, s1s, s1d, m1, srcp, dstp,
              p1parts,
              srcc, dstc, s1sv, s1dv, m1v,
              idxS0, idxS1, idxD, rows0, rows1, scaled,
              zb, sem0, sem1,
              p1acc):
    c = lax.axis_index("c")
    s = lax.axis_index("s")
    z16 = jnp.zeros((16,), jnp.float32)
    iota = lax.iota(jnp.int32, 16)

    pltpu.sync_copy(srcp.at[s], srcc)
    pltpu.sync_copy(dstp.at[s], dstc)

    def zero_zb(i, _):
        for k in range(HC // 16):
            zb[i, pl.ds(k * 16, 16)] = z16
        return _
    lax.fori_loop(0, 320, zero_zb, None)

    def issue(g, toff, idxSb, rowsb, semb):
        src16 = srcc[pl.ds(g * 16, 16)]
        idxSb[...] = src16 + toff
        @pl.when(c == 0)
        def _():
            pltpu.async_copy(xA.at[idxSb], rowsb, semb)
        @pl.when(c == 1)
        def _():
            pltpu.async_copy(xB.at[idxSb], rowsb, semb)

    def wait(idxSb, rowsb, semb):
        @pl.when(c == 0)
        def _():
            pltpu.make_async_copy(xA.at[idxSb], rowsb, semb).wait()
        @pl.when(c == 1)
        def _():
            pltpu.make_async_copy(xB.at[idxSb], rowsb, semb).wait()

    def t_body(t, _):
        pltpu.sync_copy(s1s.at[t], s1sv)
        pltpu.sync_copy(s1d.at[t], s1dv)
        pltpu.sync_copy(m1.at[t], m1v)
        m1t = m1v[...]
        toff = t * N

        for j in range(2):
            dst_off = pl.multiple_of(s * NPT + j * 320, 8)
            pltpu.sync_copy(zb, p1acc.at[pl.ds(dst_off, 320), :])
        plsc.subcore_barrier()

        issue(0, toff, idxS0, rows0, sem0)

        def process(g, idxSb, rowsb, semb):
            src16 = srcc[pl.ds(g * 16, 16)]
            dst16 = dstc[pl.ds(g * 16, 16)]
            s_s = plsc.load_gather(s1sv, [src16])
            s_d = plsc.load_gather(s1dv, [dst16])
            e = s_s + s_d
            e = jnp.where(e >= 0.0, e, e * 0.2)
            ex = jnp.exp(e - m1t)
            ex = jnp.where((g * 16 + iota) < EC, ex, 0.0)
            wait(idxSb, rowsb, semb)
            for j in range(16):
                exj = ex[j]
                for k in range(HC // 16):
                    sl = pl.ds(k * 16, 16)
                    scaled[j, sl] = rowsb[j, sl] * exj
            idxD[...] = dst16
            pltpu.sync_copy(scaled, p1acc.at[idxD], add=True)

        def pair_body(p, _):
            g0 = p * 2
            issue(g0 + 1, toff, idxS1, rows1, sem1)
            process(g0, idxS0, rows0, sem0)
            issue(g0 + 2, toff, idxS0, rows0, sem0)
            process(g0 + 1, idxS1, rows1, sem1)
            return _
        lax.fori_loop(0, NG // 2, pair_body, None)
        wait(idxS0, rows0, sem0)
        plsc.subcore_barrier()

        src_off = pl.multiple_of(s * NPT, 8)
        pltpu.sync_copy(p1acc.at[pl.ds(src_off, NPT), :],
                        p1parts.at[c, t, pl.ds(src_off, NPT), :])
        return _
    lax.fori_loop(0, T, t_body, None)


def _sc1_layer1(xA, xB, s1s, s1d, m1, srcp, dstp):
    mesh = plsc.VectorSubcoreMesh(core_axis_name="c", subcore_axis_name="s")
    f = pl.kernel(
        _sc1_body,
        out_type=jax.ShapeDtypeStruct((NC, T, NP, HC), jnp.float32),
        mesh=mesh,
        compiler_params=pltpu.CompilerParams(needs_layout_passes=False,
                                             use_tc_tiling_on_sc=False),
        scratch_types=[
            pltpu.VMEM((ECA,), jnp.int32),
            pltpu.VMEM((ECA,), jnp.int32),
            pltpu.VMEM((N,), jnp.float32),
            pltpu.VMEM((N,), jnp.float32),
            pltpu.VMEM((16,), jnp.float32),
            pltpu.VMEM((16,), jnp.int32),
            pltpu.VMEM((16,), jnp.int32),
            pltpu.VMEM((16,), jnp.int32),
            pltpu.VMEM((16, HC), jnp.float32),
            pltpu.VMEM((16, HC), jnp.float32),
            pltpu.VMEM((16, HC), jnp.float32),
            pltpu.VMEM((320, HC), jnp.float32),
            pltpu.SemaphoreType.DMA,
            pltpu.SemaphoreType.DMA,
            pltpu.VMEM_SHARED((NP, HC), jnp.float32),
        ],
    )
    return f(xA, xB, s1s, s1d, m1, srcp, dstp)


NGC = 313


def _sc2_body(h1flat, s2s, s2d, m2, srcp, dstp, bidx,
              out2parts, den2parts,
              srcc, dstc, s2sv, s2dv, m2v, bidxv, den2v,
              idxS, slotb, rows, scaled, zb2, sem,
              out2acc):
    c = lax.axis_index("c")
    s = lax.axis_index("s")
    z16 = jnp.zeros((16,), jnp.float32)
    iota = lax.iota(jnp.int32, 16)

    pltpu.sync_copy(srcp.at[s], srcc)
    pltpu.sync_copy(dstp.at[s], dstc)
    pltpu.sync_copy(bidx, bidxv)
    for k in range(16):
        for q in range(8):
            zb2[k, pl.ds(q * 16, 16)] = z16
    bvec = bidxv[...]
    base = c * (EC // 2)
    limit = base + (EC // 2)

    def t_body(t, _):
        pltpu.sync_copy(s2s.at[t], s2sv)
        pltpu.sync_copy(s2d.at[t], s2dv)
        pltpu.sync_copy(m2.at[t], m2v)
        m2t = m2v[...]
        toff = t * N

        @pl.when(s == 0)
        def _():
            pltpu.sync_copy(zb2, out2acc)
        plsc.subcore_barrier()

        def g_body(g, accs):
            off = base + g * 16
            dst16 = dstc[pl.ds(off, 16)]
            valid = (off + iota) < limit
            hit = valid
            m = jnp.zeros((16,), jnp.bool_)
            for j in range(8):
                m = m | (dst16 == bvec[j])
            hit = hit & m
            anyhit = jnp.max(jnp.where(hit, 1, 0), axis=0)

            def do_group(accs):
                src16 = srcc[pl.ds(off, 16)]
                slot = jnp.full((16,), 8, jnp.int32)
                for j in range(7, -1, -1):
                    slot = jnp.where(dst16 == bvec[j], j, slot)
                s_s = plsc.load_gather(s2sv, [src16])
                s_d = plsc.load_gather(s2dv, [slot])
                e = s_s + s_d
                e = jnp.where(e >= 0.0, e, e * 0.2)
                ex = jnp.exp(e - m2t)
                ex = jnp.where(hit, ex, 0.0)
                naccs = tuple(accs[j] + jnp.where(slot == j, ex, 0.0)
                              for j in range(8))
                idxS[...] = src16 + toff
                slotb[...] = slot
                pltpu.async_copy(h1flat.at[idxS], rows, sem).wait()
                for j in range(16):
                    exj = ex[j]
                    for k in range(8):
                        sl = pl.ds(k * 16, 16)
                        scaled[j, sl] = rows[j, sl] * exj
                pltpu.sync_copy(scaled, out2acc.at[slotb], add=True)
                return naccs

            def skip(accs):
                return accs
            return lax.cond(anyhit > 0, do_group, skip, accs)

        accs = tuple(z16 for _ in range(8))
        accs = lax.fori_loop(0, NGC, g_body, accs)
        plsc.subcore_barrier()

        for j in range(8):
            den2v[j, pl.ds(0, 16)] = accs[j]
        pltpu.sync_copy(den2v, den2parts.at[c, s, t])

        @pl.when(s == 0)
        def _():
            pltpu.sync_copy(out2acc, out2parts.at[c, t])
        return _
    lax.fori_loop(0, T, t_body, None)


def _sc2_layer2(h1flat, s2s, s2d, m2, srcp, dstp, bidx):
    mesh = plsc.VectorSubcoreMesh(core_axis_name="c", subcore_axis_name="s")
    f = pl.kernel(
        _sc2_body,
        out_type=(
            jax.ShapeDtypeStruct((NC, T, 16, H), jnp.float32),
            jax.ShapeDtypeStruct((NC, NS, T, 8, 16), jnp.float32),
        ),
        mesh=mesh,
        compiler_params=pltpu.CompilerParams(needs_layout_passes=False,
                                             use_tc_tiling_on_sc=False),
        scratch_types=[
            pltpu.VMEM((ECA,), jnp.int32),
            pltpu.VMEM((ECA,), jnp.int32),
            pltpu.VMEM((N,), jnp.float32),
            pltpu.VMEM((16,), jnp.float32),
            pltpu.VMEM((16,), jnp.float32),
            pltpu.VMEM((16,), jnp.int32),
            pltpu.VMEM((8, 16), jnp.float32),
            pltpu.VMEM((16,), jnp.int32),
            pltpu.VMEM((16,), jnp.int32),
            pltpu.VMEM((16, H), jnp.float32),
            pltpu.VMEM((16, H), jnp.float32),
            pltpu.VMEM((16, H), jnp.float32),
            pltpu.SemaphoreType.DMA,
            pltpu.VMEM_SHARED((16, H), jnp.float32),
        ],
    )
    return f(h1flat, s2s, s2d, m2, srcp, dstp, bidx)


def _lstm_head_body(series_ref, wih_ref, whh_ref, b_ref, hw_ref, hb_ref,
                    cast_ref, h_ref, c_ref):
    h = jnp.zeros((B, LH), jnp.float32)
    c = jnp.zeros((B, LH), jnp.float32)
    wih = wih_ref[...]
    whh = whh_ref[...]
    b = b_ref[...]
    hw = hw_ref[...]
    hb = hb_ref[...]
    for t in range(T):
        x_t = series_ref[t]
        z = jnp.dot(x_t, wih, preferred_element_type=jnp.float32) + \
            jnp.dot(h, whh, preferred_element_type=jnp.float32) + b
        i = jax.nn.sigmoid(z[:, 0 * LH:1 * LH])
        f = jax.nn.sigmoid(z[:, 1 * LH:2 * LH])
        g = jnp.tanh(z[:, 2 * LH:3 * LH])
        o = jax.nn.sigmoid(z[:, 3 * LH:4 * LH])
        c = f * c + i * g
        h = o * jnp.tanh(c)
        zc = jnp.dot(h, hw, preferred_element_type=jnp.float32) + hb
        m_ = zc[:, 0:K]
        b_ = jax.nn.softplus(zc[:, K:2 * K]) + 1e-5
        t_ = jax.nn.sigmoid(zc[:, 2 * K:3 * K])
        p_ = jax.nn.softmax(zc[:, 3 * K:4 * K], axis=-1)
        cast_ref[t] = jnp.concatenate([m_, b_, t_, p_], axis=-1)
    h_ref[...] = h
    c_ref[...] = c


def _lstm_head(series_tbh, W_ih, W_hh, b_lstm, head_W, head_b):
    return pl.pallas_call(
        _lstm_head_body,
        out_shape=(
            jax.ShapeDtypeStruct((T, B, 4 * K), jnp.float32),
            jax.ShapeDtypeStruct((B, LH), jnp.float32),
            jax.ShapeDtypeStruct((B, LH), jnp.float32),
        ),
    )(series_tbh, W_ih, W_hh, b_lstm, head_W, head_b)


def kernel(era5, basinContinuous, basinDiscrete, riverContinuous, riverDiscrete,
           bp_Wc, bp_bc, bp_Wd, bp_bd,
           g1_W, g1_as, g1_ad, g1_b, g2_W, g2_as, g2_ad, g2_b,
           rp_Wc, rp_bc, rp_Wd, rp_bd,
           W_ih, W_hh, b_lstm, head_W, head_b,
           edge_index, nodes):
    src, dst = edge_index[0], edge_index[1]

    W_era = bp_Wc[:D_ERA]
    W_bc = bp_Wc[D_ERA:]
    base = basinContinuous @ W_bc + basinDiscrete @ bp_Wd + (bp_bc + bp_bd)
    x_all = jax.nn.relu(jnp.einsum('ntd,dh->tnh', era5, W_era) + base[None])

    v1s = g1_W @ g1_as
    v1d = g1_W @ g1_ad
    s1s = x_all @ v1s
    s1d = x_all @ v1d
    M1 = jnp.max(s1s, axis=1) + jnp.max(s1d, axis=1)
    m1bc = jnp.broadcast_to(M1[:, None], (T, 16))

    srcp = jnp.pad(src.reshape(NS, EC), ((0, 0), (0, ECA - EC)))
    dstp = jnp.pad(dst.reshape(NS, EC), ((0, 0), (0, ECA - EC)))
    xflat = x_all.reshape(T * N, H)
    zc = jnp.zeros((T * N, 16), jnp.float32)
    xA = jnp.concatenate([xflat[:, :64], zc], axis=1)
    xB = jnp.concatenate([xflat[:, 64:], jnp.ones((T * N, 1), jnp.float32),
                          zc[:, :15]], axis=1)
    p1parts = _sc1_layer1(xA, xB, s1s, s1d, m1bc, srcp, dstp)
    P1 = jnp.concatenate([p1parts[0, :, :N, :64], p1parts[1, :, :N, :64]], -1)
    denom = p1parts[1, :, :N, 64]
    h1 = jax.nn.elu((P1 / (denom[..., None] + 1e-16)) @ g1_W + g1_b)

    batchIndices = jnp.concatenate([jnp.zeros((1,), nodes.dtype), jnp.cumsum(nodes)[:-1]])

    v2s = g2_W @ g2_as
    v2d = g2_W @ g2_ad
    s2s = h1 @ v2s
    s2d_sel = h1[:, batchIndices, :] @ v2d
    M2 = jnp.max(s2s, axis=1) + jnp.max(s2d_sel, axis=1)

    s2d_pad = jnp.pad(s2d_sel, ((0, 0), (0, 8)))
    m2bc = jnp.broadcast_to(M2[:, None], (T, 16))
    bidx_pad = jnp.pad(batchIndices.astype(jnp.int32), (0, 8),
                       constant_values=-1)
    out2parts, den2parts = _sc2_layer2(h1.reshape(T * N, H), s2s, s2d_pad,
                                       m2bc, srcp, dstp, bidx_pad)
    agg = out2parts.sum(axis=0)[:, :8, :]
    denom2 = den2parts.sum(axis=(0, 1, 4))
    out2 = (agg / (denom2[..., None] + 1e-16)) @ g2_W + g2_b
    first = jnp.argmax(batchIndices[None, :] == batchIndices[:, None], axis=1)
    out2 = out2[:, first, :]

    rcat = jnp.concatenate([out2, jnp.broadcast_to(riverContinuous[None], (T, B, D_RC))], -1)
    series = jax.nn.relu(rcat @ rp_Wc + rp_bc + riverDiscrete @ rp_Wd + rp_bd)

    cast_t, h, c = _lstm_head(series, W_ih, W_hh, b_lstm, head_W, head_b)
    cast = jnp.swapaxes(cast_t, 0, 1)
    return cast, (h, c)


# speedup vs baseline: 34.6395x; 1.2988x over previous
import functools
import jax
import jax.numpy as jnp
from jax import lax
from jax.experimental import pallas as pl
from jax.experimental.pallas import tpu as pltpu
from jax.experimental.pallas import tpu_sc as plsc

N = 10000; T = 16; E = 160000; B = 8
D_ERA = 16; D_BC = 32; D_BD = 16; D_RC = 16; D_RD = 8
H = 128; LH = 256; K = 3

NC = 2
NS = 16
EC = E // NS
ECP = 10048
NG = ECP // 16
ECA = ECP + 64
HC = 80
NP = 10240
NPT = NP // NS


def _sc1_body(xA, xB, s1s, s1d, m1, srcp, dstp,
              p1parts,
              srcc, dstc, s1sv, s1dv, m1v,
              idxS, idxD, rows, scaled,
              zb, gsems, ssems,
              p1acc):
    c = lax.axis_index("c")
    s = lax.axis_index("s")
    z16 = jnp.zeros((16,), jnp.float32)
    iota = lax.iota(jnp.int32, 16)
    D = 4

    pltpu.sync_copy(srcp.at[s], srcc)
    pltpu.sync_copy(dstp.at[s], dstc)

    def zero_zb(i, _):
        for k in range(HC // 16):
            zb[i, pl.ds(k * 16, 16)] = z16
        return _
    lax.fori_loop(0, 320, zero_zb, None)

    def gissue(g, toff, b):
        src16 = srcc[pl.ds(g * 16, 16)]
        idxS[b, pl.ds(0, 16)] = src16 + toff
        @pl.when(c == 0)
        def _():
            pltpu.async_copy(xA.at[idxS.at[b]], rows.at[b], gsems.at[b])
        @pl.when(c == 1)
        def _():
            pltpu.async_copy(xB.at[idxS.at[b]], rows.at[b], gsems.at[b])

    def gwait(b):
        @pl.when(c == 0)
        def _():
            pltpu.make_async_copy(xA.at[idxS.at[b]], rows.at[b],
                                  gsems.at[b]).wait()
        @pl.when(c == 1)
        def _():
            pltpu.make_async_copy(xB.at[idxS.at[b]], rows.at[b],
                                  gsems.at[b]).wait()

    def swait(b):
        pltpu.make_async_copy(scaled.at[b], p1acc.at[idxD.at[b]],
                              ssems.at[b]).wait()

    def t_body(t, _):
        pltpu.sync_copy(s1s.at[t], s1sv)
        pltpu.sync_copy(s1d.at[t], s1dv)
        pltpu.sync_copy(m1.at[t], m1v)
        m1t = m1v[...]
        toff = t * N

        for j in range(2):
            dst_off = pl.multiple_of(s * NPT + j * 320, 8)
            pltpu.sync_copy(zb, p1acc.at[pl.ds(dst_off, 320), :])
        plsc.subcore_barrier()

        for b in range(D):
            gissue(b, toff, b)

        def quad_body(q, _):
            for b in range(D):
                g = q * D + b
                src16 = srcc[pl.ds(g * 16, 16)]
                dst16 = dstc[pl.ds(g * 16, 16)]
                s_s = plsc.load_gather(s1sv, [src16])
                s_d = plsc.load_gather(s1dv, [dst16])
                e = s_s + s_d
                e = jnp.where(e >= 0.0, e, e * 0.2)
                ex = jnp.exp(e - m1t)
                ex = jnp.where((g * 16 + iota) < EC, ex, 0.0)
                gwait(b)
                @pl.when(q > 0)
                def _():
                    swait(b)
                for j in range(16):
                    exj = ex[j]
                    for k in range(HC // 16):
                        sl = pl.ds(k * 16, 16)
                        scaled[b, j, sl] = rows[b, j, sl] * exj
                idxD[b, pl.ds(0, 16)] = dst16
                pltpu.async_copy(scaled.at[b], p1acc.at[idxD.at[b]],
                                 ssems.at[b], add=True)
                gissue(g + D, toff, b)
            return _
        lax.fori_loop(0, NG // D, quad_body, None)
        for b in range(D):
            gwait(b)
            swait(b)
        plsc.subcore_barrier()

        src_off = pl.multiple_of(s * NPT, 8)
        pltpu.sync_copy(p1acc.at[pl.ds(src_off, NPT), :],
                        p1parts.at[c, t, pl.ds(src_off, NPT), :])
        return _
    lax.fori_loop(0, T, t_body, None)


def _sc1_layer1(xA, xB, s1s, s1d, m1, srcp, dstp):
    mesh = plsc.VectorSubcoreMesh(core_axis_name="c", subcore_axis_name="s")
    f = pl.kernel(
        _sc1_body,
        out_type=jax.ShapeDtypeStruct((NC, T, NP, HC), jnp.float32),
        mesh=mesh,
        compiler_params=pltpu.CompilerParams(needs_layout_passes=False,
                                             use_tc_tiling_on_sc=False),
        scratch_types=[
            pltpu.VMEM((ECA,), jnp.int32),
            pltpu.VMEM((ECA,), jnp.int32),
            pltpu.VMEM((N,), jnp.float32),
            pltpu.VMEM((N,), jnp.float32),
            pltpu.VMEM((16,), jnp.float32),
            pltpu.VMEM((4, 16), jnp.int32),
            pltpu.VMEM((4, 16), jnp.int32),
            pltpu.VMEM((4, 16, HC), jnp.float32),
            pltpu.VMEM((4, 16, HC), jnp.float32),
            pltpu.VMEM((320, HC), jnp.float32),
            pltpu.SemaphoreType.DMA((4,)),
            pltpu.SemaphoreType.DMA((4,)),
            pltpu.VMEM_SHARED((NP, HC), jnp.float32),
        ],
    )
    return f(xA, xB, s1s, s1d, m1, srcp, dstp)


NGC = 313


def _sc2_body(h1flat, s2s, s2d, m2, srcp, dstp, bidx,
              out2parts, den2parts,
              srcc, dstc, s2sv, s2dv, m2v, bidxv, den2v,
              idxS, slotb, rows, scaled, zb2, sem,
              out2acc):
    c = lax.axis_index("c")
    s = lax.axis_index("s")
    z16 = jnp.zeros((16,), jnp.float32)
    iota = lax.iota(jnp.int32, 16)

    pltpu.sync_copy(srcp.at[s], srcc)
    pltpu.sync_copy(dstp.at[s], dstc)
    pltpu.sync_copy(bidx, bidxv)
    for k in range(16):
        for q in range(8):
            zb2[k, pl.ds(q * 16, 16)] = z16
    bvec = bidxv[...]
    base = c * (EC // 2)
    limit = base + (EC // 2)

    def t_body(t, _):
        pltpu.sync_copy(s2s.at[t], s2sv)
        pltpu.sync_copy(s2d.at[t], s2dv)
        pltpu.sync_copy(m2.at[t], m2v)
        m2t = m2v[...]
        toff = t * N

        @pl.when(s == 0)
        def _():
            pltpu.sync_copy(zb2, out2acc)
        plsc.subcore_barrier()

        def g_body(g, accs):
            off = base + g * 16
            dst16 = dstc[pl.ds(off, 16)]
            valid = (off + iota) < limit
            hit = valid
            m = jnp.zeros((16,), jnp.bool_)
            for j in range(8):
                m = m | (dst16 == bvec[j])
            hit = hit & m
            anyhit = jnp.max(jnp.where(hit, 1, 0), axis=0)

            def do_group(accs):
                src16 = srcc[pl.ds(off, 16)]
                slot = jnp.full((16,), 8, jnp.int32)
                for j in range(7, -1, -1):
                    slot = jnp.where(dst16 == bvec[j], j, slot)
                s_s = plsc.load_gather(s2sv, [src16])
                s_d = plsc.load_gather(s2dv, [slot])
                e = s_s + s_d
                e = jnp.where(e >= 0.0, e, e * 0.2)
                ex = jnp.exp(e - m2t)
                ex = jnp.where(hit, ex, 0.0)
                naccs = tuple(accs[j] + jnp.where(slot == j, ex, 0.0)
                              for j in range(8))
                idxS[...] = src16 + toff
                slotb[...] = slot
                pltpu.async_copy(h1flat.at[idxS], rows, sem).wait()
                for j in range(16):
                    exj = ex[j]
                    for k in range(8):
                        sl = pl.ds(k * 16, 16)
                        scaled[j, sl] = rows[j, sl] * exj
                pltpu.sync_copy(scaled, out2acc.at[slotb], add=True)
                return naccs

            def skip(accs):
                return accs
            return lax.cond(anyhit > 0, do_group, skip, accs)

        accs = tuple(z16 for _ in range(8))
        accs = lax.fori_loop(0, NGC, g_body, accs)
        plsc.subcore_barrier()

        for j in range(8):
            den2v[j, pl.ds(0, 16)] = accs[j]
        pltpu.sync_copy(den2v, den2parts.at[c, s, t])

        @pl.when(s == 0)
        def _():
            pltpu.sync_copy(out2acc, out2parts.at[c, t])
        return _
    lax.fori_loop(0, T, t_body, None)


def _sc2_layer2(h1flat, s2s, s2d, m2, srcp, dstp, bidx):
    mesh = plsc.VectorSubcoreMesh(core_axis_name="c", subcore_axis_name="s")
    f = pl.kernel(
        _sc2_body,
        out_type=(
            jax.ShapeDtypeStruct((NC, T, 16, H), jnp.float32),
            jax.ShapeDtypeStruct((NC, NS, T, 8, 16), jnp.float32),
        ),
        mesh=mesh,
        compiler_params=pltpu.CompilerParams(needs_layout_passes=False,
                                             use_tc_tiling_on_sc=False),
        scratch_types=[
            pltpu.VMEM((ECA,), jnp.int32),
            pltpu.VMEM((ECA,), jnp.int32),
            pltpu.VMEM((N,), jnp.float32),
            pltpu.VMEM((16,), jnp.float32),
            pltpu.VMEM((16,), jnp.float32),
            pltpu.VMEM((16,), jnp.int32),
            pltpu.VMEM((8, 16), jnp.float32),
            pltpu.VMEM((16,), jnp.int32),
            pltpu.VMEM((16,), jnp.int32),
            pltpu.VMEM((16, H), jnp.float32),
            pltpu.VMEM((16, H), jnp.float32),
            pltpu.VMEM((16, H), jnp.float32),
            pltpu.SemaphoreType.DMA,
            pltpu.VMEM_SHARED((16, H), jnp.float32),
        ],
    )
    return f(h1flat, s2s, s2d, m2, srcp, dstp, bidx)


def _lstm_head_body(series_ref, wih_ref, whh_ref, b_ref, hw_ref, hb_ref,
                    cast_ref, h_ref, c_ref):
    h = jnp.zeros((B, LH), jnp.float32)
    c = jnp.zeros((B, LH), jnp.float32)
    wih = wih_ref[...]
    whh = whh_ref[...]
    b = b_ref[...]
    hw = hw_ref[...]
    hb = hb_ref[...]
    for t in range(T):
        x_t = series_ref[t]
        z = jnp.dot(x_t, wih, preferred_element_type=jnp.float32) + \
            jnp.dot(h, whh, preferred_element_type=jnp.float32) + b
        i = jax.nn.sigmoid(z[:, 0 * LH:1 * LH])
        f = jax.nn.sigmoid(z[:, 1 * LH:2 * LH])
        g = jnp.tanh(z[:, 2 * LH:3 * LH])
        o = jax.nn.sigmoid(z[:, 3 * LH:4 * LH])
        c = f * c + i * g
        h = o * jnp.tanh(c)
        zc = jnp.dot(h, hw, preferred_element_type=jnp.float32) + hb
        m_ = zc[:, 0:K]
        b_ = jax.nn.softplus(zc[:, K:2 * K]) + 1e-5
        t_ = jax.nn.sigmoid(zc[:, 2 * K:3 * K])
        p_ = jax.nn.softmax(zc[:, 3 * K:4 * K], axis=-1)
        cast_ref[t] = jnp.concatenate([m_, b_, t_, p_], axis=-1)
    h_ref[...] = h
    c_ref[...] = c


def _lstm_head(series_tbh, W_ih, W_hh, b_lstm, head_W, head_b):
    return pl.pallas_call(
        _lstm_head_body,
        out_shape=(
            jax.ShapeDtypeStruct((T, B, 4 * K), jnp.float32),
            jax.ShapeDtypeStruct((B, LH), jnp.float32),
            jax.ShapeDtypeStruct((B, LH), jnp.float32),
        ),
    )(series_tbh, W_ih, W_hh, b_lstm, head_W, head_b)


def kernel(era5, basinContinuous, basinDiscrete, riverContinuous, riverDiscrete,
           bp_Wc, bp_bc, bp_Wd, bp_bd,
           g1_W, g1_as, g1_ad, g1_b, g2_W, g2_as, g2_ad, g2_b,
           rp_Wc, rp_bc, rp_Wd, rp_bd,
           W_ih, W_hh, b_lstm, head_W, head_b,
           edge_index, nodes):
    src, dst = edge_index[0], edge_index[1]

    W_era = bp_Wc[:D_ERA]
    W_bc = bp_Wc[D_ERA:]
    base = basinContinuous @ W_bc + basinDiscrete @ bp_Wd + (bp_bc + bp_bd)
    x_all = jax.nn.relu(jnp.einsum('ntd,dh->tnh', era5, W_era) + base[None])

    v1s = g1_W @ g1_as
    v1d = g1_W @ g1_ad
    s1s = x_all @ v1s
    s1d = x_all @ v1d
    M1 = jnp.max(s1s, axis=1) + jnp.max(s1d, axis=1)
    m1bc = jnp.broadcast_to(M1[:, None], (T, 16))

    srcp = jnp.pad(src.reshape(NS, EC), ((0, 0), (0, ECA - EC)))
    dstp = jnp.pad(dst.reshape(NS, EC), ((0, 0), (0, ECA - EC)))
    xflat = x_all.reshape(T * N, H)
    zc = jnp.zeros((T * N, 16), jnp.float32)
    xA = jnp.concatenate([xflat[:, :64], zc], axis=1)
    xB = jnp.concatenate([xflat[:, 64:], jnp.ones((T * N, 1), jnp.float32),
                          zc[:, :15]], axis=1)
    p1parts = _sc1_layer1(xA, xB, s1s, s1d, m1bc, srcp, dstp)
    P1 = jnp.concatenate([p1parts[0, :, :N, :64], p1parts[1, :, :N, :64]], -1)
    denom = p1parts[1, :, :N, 64]
    h1 = jax.nn.elu((P1 / (denom[..., None] + 1e-16)) @ g1_W + g1_b)

    batchIndices = jnp.concatenate([jnp.zeros((1,), nodes.dtype), jnp.cumsum(nodes)[:-1]])

    v2s = g2_W @ g2_as
    v2d = g2_W @ g2_ad
    s2s = h1 @ v2s
    s2d_sel = h1[:, batchIndices, :] @ v2d
    M2 = jnp.max(s2s, axis=1) + jnp.max(s2d_sel, axis=1)

    s2d_pad = jnp.pad(s2d_sel, ((0, 0), (0, 8)))
    m2bc = jnp.broadcast_to(M2[:, None], (T, 16))
    bidx_pad = jnp.pad(batchIndices.astype(jnp.int32), (0, 8),
                       constant_values=-1)
    out2parts, den2parts = _sc2_layer2(h1.reshape(T * N, H), s2s, s2d_pad,
                                       m2bc, srcp, dstp, bidx_pad)
    agg = out2parts.sum(axis=0)[:, :8, :]
    denom2 = den2parts.sum(axis=(0, 1, 4))
    out2 = (agg / (denom2[..., None] + 1e-16)) @ g2_W + g2_b
    first = jnp.argmax(batchIndices[None, :] == batchIndices[:, None], axis=1)
    out2 = out2[:, first, :]

    rcat = jnp.concatenate([out2, jnp.broadcast_to(riverContinuous[None], (T, B, D_RC))], -1)
    series = jax.nn.relu(rcat @ rp_Wc + rp_bc + riverDiscrete @ rp_Wd + rp_bd)

    cast_t, h, c = _lstm_head(series, W_ih, W_hh, b_lstm, head_W, head_b)
    cast = jnp.swapaxes(cast_t, 0, 1)
    return cast, (h, c)
